# baseline (speedup 1.0000x reference)
import functools

import jax
import jax.numpy as jnp
from jax import lax
from jax.experimental import pallas as pl
from jax.experimental.pallas import tpu as pltpu
from jax.experimental.pallas import tpu_sc as plsc

NC = 2
NS = 16
LANES = 16
CB = 80


def _spmm_sc(feat, edges3, ev2, nch0, nch1):
    n_nodes, d = feat.shape
    rows_per_sub = (n_nodes // (NS * 8)) * 8
    rem = n_nodes - NS * rows_per_sub
    rstg = 64

    def _spans(length):
        out, off = [], 0
        while off < length:
            c = min(rstg, length - off)
            out.append((off, c))
            off += c
        return out

    mesh = plsc.VectorSubcoreMesh(core_axis_name="c", subcore_axis_name="s")

    @functools.partial(
        pl.kernel,
        out_type=jax.ShapeDtypeStruct((NC, n_nodes, d), jnp.float32),
        mesh=mesh,
        scratch_types=[
            pltpu.VMEM_SHARED((n_nodes, d), jnp.float32),
            pltpu.VMEM((2, 2, CB), jnp.int32),
            pltpu.VMEM((2, CB), jnp.float32),
            pltpu.VMEM((2, CB, d), jnp.float32),
            pltpu.VMEM((2, CB), jnp.int32),
            pltpu.VMEM((rstg, d), jnp.float32),
            pltpu.SemaphoreType.DMA,
            pltpu.SemaphoreType.DMA,
            pltpu.SemaphoreType.DMA,
        ],
    )
    def spmm(feat_hbm, edges_hbm, ev_hbm, out_hbm,
             acc, ebuf, evb, gbuf, sbuf, zstg, esem, gsem, ssem):
        cid = lax.axis_index("c")
        sid = lax.axis_index("s")
        nch = jnp.where(cid == 0, nch0, nch1)
        cbase = jnp.where(cid == 0, sid * nch0, NS * nch0 + sid * nch1)

        zeros16 = jnp.zeros((LANES,), jnp.float32)

        def zb(i, carry):
            for j in range(d // LANES):
                zstg[i, pl.ds(j * LANES, LANES)] = zeros16
            return carry

        lax.fori_loop(0, rstg, zb, 0)
        for off, c in _spans(rows_per_sub):
            start = pl.multiple_of(sid * rows_per_sub + off, 8)
            pltpu.sync_copy(zstg.at[pl.ds(0, c)], acc.at[pl.ds(start, c)])
        if rem:
            @pl.when(sid == NS - 1)
            def _():
                for off, c in _spans(rem):
                    pltpu.sync_copy(
                        zstg.at[pl.ds(0, c)],
                        acc.at[pl.ds(NS * rows_per_sub + off, c)])

        plsc.subcore_barrier()

        pltpu.sync_copy(edges_hbm.at[cbase], ebuf.at[0])
        pltpu.sync_copy(ev_hbm.at[cbase], evb.at[0])
        pltpu.async_copy(feat_hbm.at[ebuf.at[0, 1]], gbuf.at[0], gsem)
        if min(nch0, nch1) > 1:
            pltpu.async_copy(edges_hbm.at[cbase + 1], ebuf.at[1], esem)
            pltpu.async_copy(ev_hbm.at[cbase + 1], evb.at[1], esem)

        def chunk2(it, carry):
            for par in (0, 1):
                ch = it * 2 + par
                nxt = 1 - par

                @pl.when(ch + 1 < nch)
                def _(ch=ch, par=par, nxt=nxt):
                    pltpu.make_async_copy(edges_hbm.at[cbase + ch + 1],
                                          ebuf.at[nxt], esem).wait()
                    pltpu.make_async_copy(ev_hbm.at[cbase + ch + 1],
                                          evb.at[nxt], esem).wait()

                    @pl.when(ch >= 1)
                    def _():
                        pltpu.make_async_copy(
                            gbuf.at[nxt], acc.at[sbuf.at[nxt]], ssem).wait()

                    pltpu.async_copy(feat_hbm.at[ebuf.at[nxt, 1]],
                                     gbuf.at[nxt], gsem)

                pltpu.make_async_copy(feat_hbm.at[ebuf.at[par, 1]],
                                      gbuf.at[par], gsem).wait()

                def scale(g, c2, par=par):
                    wv = evb[par, pl.ds(g * LANES, LANES)]
                    for l in range(LANES):
                        w = wv[l]
                        e = g * LANES + l
                        for j in range(d // LANES):
                            sl = pl.ds(j * LANES, LANES)
                            gbuf[par, e, sl] = gbuf[par, e, sl] * w
                    return c2

                lax.fori_loop(0, CB // LANES, scale, 0)

                for g in range(CB // LANES):
                    sl = pl.ds(g * LANES, LANES)
                    sbuf[par, sl] = ebuf[par, 0, sl]
                pltpu.async_copy(gbuf.at[par], acc.at[sbuf.at[par]], ssem,
                                 add=True)

                @pl.when(ch + 2 < nch)
                def _(ch=ch, par=par):
                    pltpu.async_copy(edges_hbm.at[cbase + ch + 2],
                                     ebuf.at[par], esem)
                    pltpu.async_copy(ev_hbm.at[cbase + ch + 2],
                                     evb.at[par], esem)

            return carry

        lax.fori_loop(0, nch // 2, chunk2, 0)
        for par in (0, 1):
            pltpu.make_async_copy(gbuf.at[par], acc.at[sbuf.at[par]],
                                  ssem).wait()

        plsc.subcore_barrier()

        for off, c in _spans(rows_per_sub):
            start = pl.multiple_of(sid * rows_per_sub + off, 8)
            pltpu.sync_copy(acc.at[pl.ds(start, c)], zstg.at[pl.ds(0, c)])
            pltpu.sync_copy(zstg.at[pl.ds(0, c)],
                            out_hbm.at[cid, pl.ds(start, c)])
        if rem:
            @pl.when(sid == NS - 1)
            def _():
                for off, c in _spans(rem):
                    start = NS * rows_per_sub + off
                    pltpu.sync_copy(acc.at[pl.ds(start, c)],
                                    zstg.at[pl.ds(0, c)])
                    pltpu.sync_copy(zstg.at[pl.ds(0, c)],
                                    out_hbm.at[cid, pl.ds(start, c)])

    return spmm(feat, edges3, ev2)


def _normalize_tc(x):
    n_nodes, d = x.shape
    bm = 2000

    def body(x_ref, o_ref):
        xb = x_ref[...]
        s = jnp.sum(xb, axis=1, keepdims=True) + 0.0001
        o_ref[...] = xb / s

    return pl.pallas_call(
        body,
        grid=(n_nodes // bm,),
        in_specs=[pl.BlockSpec((bm, d), lambda i: (i, 0))],
        out_specs=pl.BlockSpec((bm, d), lambda i: (i, 0)),
        out_shape=jax.ShapeDtypeStruct((n_nodes, d), jnp.float32),
    )(x)


def _fused_linear_tc(partials, w, b, relu):
    _, n_nodes, d = partials.shape
    bm = 2000

    def body(p_ref, w_ref, b_ref, o_ref):
        a = p_ref[0] + p_ref[1]
        y = lax.dot_general(a, w_ref[...], (((1,), (1,)), ((), ())),
                            preferred_element_type=jnp.float32) + b_ref[...]
        o_ref[...] = jnp.maximum(y, 0.0) if relu else y

    return pl.pallas_call(
        body,
        grid=(n_nodes // bm,),
        in_specs=[pl.BlockSpec((NC, bm, d), lambda i: (0, i, 0)),
                  pl.BlockSpec((d, d), lambda i: (0, 0)),
                  pl.BlockSpec((1, d), lambda i: (0, 0))],
        out_specs=pl.BlockSpec((bm, d), lambda i: (i, 0)),
        out_shape=jax.ShapeDtypeStruct((n_nodes, d), jnp.float32),
    )(partials, w, b.reshape(1, d))


def kernel(x, edge_index, edge_values, W1, b1, W2, b2):
    n_nodes, d = x.shape
    n_edges = edge_index.shape[1]
    step = 2 * NS * CB
    e_pad = ((n_edges + step - 1) // step) * step
    row = edge_index[0]
    col = edge_index[1]
    ev = edge_values
    if e_pad != n_edges:
        pad = e_pad - n_edges
        row = jnp.concatenate([row, jnp.zeros((pad,), jnp.int32)])
        col = jnp.concatenate([col, jnp.zeros((pad,), jnp.int32)])
        ev = jnp.concatenate([ev, jnp.zeros((pad,), jnp.float32)])
    t_chunks = e_pad // CB
    nch_sum = t_chunks // NS
    frac0 = 0.30
    nch0 = max(2, int(round(nch_sum * frac0 / 2)) * 2)
    nch1 = nch_sum - nch0
    edges3 = jnp.stack([row.reshape(t_chunks, CB),
                        col.reshape(t_chunks, CB)], axis=1)
    ev2 = ev.reshape(t_chunks, CB)

    xn = _normalize_tc(x)
    p1 = _spmm_sc(xn, edges3, ev2, nch0, nch1)
    h = _fused_linear_tc(p1, W1, b1, relu=True)
    p2 = _spmm_sc(h, edges3, ev2, nch0, nch1)
    y = _fused_linear_tc(p2, W2, b2, relu=False)
    return y


# speedup vs baseline: 1.1795x; 1.1795x over previous
import functools

import jax
import jax.numpy as jnp
from jax import lax
from jax.experimental import pallas as pl
from jax.experimental.pallas import tpu as pltpu
from jax.experimental.pallas import tpu_sc as plsc

NC = 2
NS = 16
LANES = 16
CB = 80


def _spmm_sc(feat, edges3, ev2, nch0, nch1):
    n_nodes, d = feat.shape
    rows_per_sub = (n_nodes // (NS * 8)) * 8
    rem = n_nodes - NS * rows_per_sub
    rstg = 64

    def _spans(length):
        out, off = [], 0
        while off < length:
            c = min(rstg, length - off)
            out.append((off, c))
            off += c
        return out

    mesh = plsc.VectorSubcoreMesh(core_axis_name="c", subcore_axis_name="s")

    @functools.partial(
        pl.kernel,
        out_type=jax.ShapeDtypeStruct((NC, n_nodes, d), jnp.float32),
        mesh=mesh,
        scratch_types=[
            pltpu.VMEM_SHARED((n_nodes, d), jnp.float32),
            pltpu.VMEM((2, 2, CB), jnp.int32),
            pltpu.VMEM((2, CB), jnp.float32),
            pltpu.VMEM((2, CB, d), jnp.float32),
            pltpu.VMEM((2, CB), jnp.int32),
            pltpu.VMEM((rstg, d), jnp.float32),
            pltpu.SemaphoreType.DMA,
            pltpu.SemaphoreType.DMA,
            pltpu.SemaphoreType.DMA,
        ],
    )
    def spmm(feat_hbm, edges_hbm, ev_hbm, out_hbm,
             acc, ebuf, evb, gbuf, sbuf, zstg, esem, gsem, ssem):
        cid = lax.axis_index("c")
        sid = lax.axis_index("s")
        nch = jnp.where(cid == 0, nch0, nch1)
        cbase = jnp.where(cid == 0, sid * nch0, NS * nch0 + sid * nch1)

        zeros16 = jnp.zeros((LANES,), jnp.float32)

        def zb(i, carry):
            for j in range(d // LANES):
                zstg[i, pl.ds(j * LANES, LANES)] = zeros16
            return carry

        lax.fori_loop(0, rstg, zb, 0)
        for off, c in _spans(rows_per_sub):
            start = pl.multiple_of(sid * rows_per_sub + off, 8)
            pltpu.sync_copy(zstg.at[pl.ds(0, c)], acc.at[pl.ds(start, c)])
        if rem:
            @pl.when(sid == NS - 1)
            def _():
                for off, c in _spans(rem):
                    pltpu.sync_copy(
                        zstg.at[pl.ds(0, c)],
                        acc.at[pl.ds(NS * rows_per_sub + off, c)])

        plsc.subcore_barrier()

        pltpu.sync_copy(edges_hbm.at[cbase], ebuf.at[0])
        pltpu.sync_copy(ev_hbm.at[cbase], evb.at[0])
        pltpu.async_copy(feat_hbm.at[ebuf.at[0, 1]], gbuf.at[0], gsem)
        if min(nch0, nch1) > 1:
            pltpu.async_copy(edges_hbm.at[cbase + 1], ebuf.at[1], esem)
            pltpu.async_copy(ev_hbm.at[cbase + 1], evb.at[1], esem)

        def chunk2(it, carry):
            for par in (0, 1):
                ch = it * 2 + par
                nxt = 1 - par

                @pl.when(ch + 1 < nch)
                def _(ch=ch, par=par, nxt=nxt):
                    pltpu.make_async_copy(edges_hbm.at[cbase + ch + 1],
                                          ebuf.at[nxt], esem).wait()
                    pltpu.make_async_copy(ev_hbm.at[cbase + ch + 1],
                                          evb.at[nxt], esem).wait()

                    @pl.when(ch >= 1)
                    def _():
                        pltpu.make_async_copy(
                            gbuf.at[nxt], acc.at[sbuf.at[nxt]], ssem).wait()

                    pltpu.async_copy(feat_hbm.at[ebuf.at[nxt, 1]],
                                     gbuf.at[nxt], gsem)

                pltpu.make_async_copy(feat_hbm.at[ebuf.at[par, 1]],
                                      gbuf.at[par], gsem).wait()

                def scale(g, c2, par=par):
                    wv = evb[par, pl.ds(g * LANES, LANES)]
                    for l in range(LANES):
                        w = wv[l]
                        e = g * LANES + l
                        for j in range(d // LANES):
                            sl = pl.ds(j * LANES, LANES)
                            gbuf[par, e, sl] = gbuf[par, e, sl] * w
                    return c2

                lax.fori_loop(0, CB // LANES, scale, 0)

                for g in range(CB // LANES):
                    sl = pl.ds(g * LANES, LANES)
                    sbuf[par, sl] = ebuf[par, 0, sl]
                pltpu.async_copy(gbuf.at[par], acc.at[sbuf.at[par]], ssem,
                                 add=True)

                @pl.when(ch + 2 < nch)
                def _(ch=ch, par=par):
                    pltpu.async_copy(edges_hbm.at[cbase + ch + 2],
                                     ebuf.at[par], esem)
                    pltpu.async_copy(ev_hbm.at[cbase + ch + 2],
                                     evb.at[par], esem)

            return carry

        lax.fori_loop(0, nch // 2, chunk2, 0)
        for par in (0, 1):
            pltpu.make_async_copy(gbuf.at[par], acc.at[sbuf.at[par]],
                                  ssem).wait()

        plsc.subcore_barrier()

        for off, c in _spans(rows_per_sub):
            start = pl.multiple_of(sid * rows_per_sub + off, 8)
            pltpu.sync_copy(acc.at[pl.ds(start, c)], zstg.at[pl.ds(0, c)])
            pltpu.sync_copy(zstg.at[pl.ds(0, c)],
                            out_hbm.at[cid, pl.ds(start, c)])
        if rem:
            @pl.when(sid == NS - 1)
            def _():
                for off, c in _spans(rem):
                    start = NS * rows_per_sub + off
                    pltpu.sync_copy(acc.at[pl.ds(start, c)],
                                    zstg.at[pl.ds(0, c)])
                    pltpu.sync_copy(zstg.at[pl.ds(0, c)],
                                    out_hbm.at[cid, pl.ds(start, c)])

    return spmm(feat, edges3, ev2)


def _normalize_tc(x):
    n_nodes, d = x.shape
    bm = 2000

    def body(x_ref, o_ref):
        xb = x_ref[...]
        s = jnp.sum(xb, axis=1, keepdims=True) + 0.0001
        o_ref[...] = xb / s

    return pl.pallas_call(
        body,
        grid=(n_nodes // bm,),
        in_specs=[pl.BlockSpec((bm, d), lambda i: (i, 0))],
        out_specs=pl.BlockSpec((bm, d), lambda i: (i, 0)),
        out_shape=jax.ShapeDtypeStruct((n_nodes, d), jnp.float32),
    )(x)


def _fused_linear_tc(partials, w, b, relu):
    _, n_nodes, d = partials.shape
    bm = 2000

    def body(p_ref, w_ref, b_ref, o_ref):
        a = p_ref[0] + p_ref[1]
        y = lax.dot_general(a, w_ref[...], (((1,), (1,)), ((), ())),
                            preferred_element_type=jnp.float32) + b_ref[...]
        o_ref[...] = jnp.maximum(y, 0.0) if relu else y

    return pl.pallas_call(
        body,
        grid=(n_nodes // bm,),
        in_specs=[pl.BlockSpec((NC, bm, d), lambda i: (0, i, 0)),
                  pl.BlockSpec((d, d), lambda i: (0, 0)),
                  pl.BlockSpec((1, d), lambda i: (0, 0))],
        out_specs=pl.BlockSpec((bm, d), lambda i: (i, 0)),
        out_shape=jax.ShapeDtypeStruct((n_nodes, d), jnp.float32),
    )(partials, w, b.reshape(1, d))


def kernel(x, edge_index, edge_values, W1, b1, W2, b2):
    n_nodes, d = x.shape
    n_edges = edge_index.shape[1]
    step = 2 * NS * CB
    e_pad = ((n_edges + step - 1) // step) * step
    row = edge_index[0]
    col = edge_index[1]
    ev = edge_values
    if e_pad != n_edges:
        pad = e_pad - n_edges
        row = jnp.concatenate([row, jnp.zeros((pad,), jnp.int32)])
        col = jnp.concatenate([col, jnp.zeros((pad,), jnp.int32)])
        ev = jnp.concatenate([ev, jnp.zeros((pad,), jnp.float32)])
    t_chunks = e_pad // CB
    nch_sum = t_chunks // NS
    frac0 = 0.44
    nch0 = max(2, int(round(nch_sum * frac0 / 2)) * 2)
    nch1 = nch_sum - nch0
    edges3 = jnp.stack([row.reshape(t_chunks, CB),
                        col.reshape(t_chunks, CB)], axis=1)
    ev2 = ev.reshape(t_chunks, CB)

    xn = _normalize_tc(x)
    p1 = _spmm_sc(xn, edges3, ev2, nch0, nch1)
    h = _fused_linear_tc(p1, W1, b1, relu=True)
    p2 = _spmm_sc(h, edges3, ev2, nch0, nch1)
    y = _fused_linear_tc(p2, W2, b2, relu=False)
    return y


# speedup vs baseline: 1.2433x; 1.0541x over previous
import functools

import jax
import jax.numpy as jnp
from jax import lax
from jax.experimental import pallas as pl
from jax.experimental.pallas import tpu as pltpu
from jax.experimental.pallas import tpu_sc as plsc

NC = 2
NS = 16
LANES = 16
CB = 80


def _spmm_sc(feat, edges3, ev2, nch0, nch1):
    n_nodes, d = feat.shape
    rows_per_sub = (n_nodes // (NS * 8)) * 8
    rem = n_nodes - NS * rows_per_sub
    rstg = 64

    def _spans(length):
        out, off = [], 0
        while off < length:
            c = min(rstg, length - off)
            out.append((off, c))
            off += c
        return out

    mesh = plsc.VectorSubcoreMesh(core_axis_name="c", subcore_axis_name="s")

    @functools.partial(
        pl.kernel,
        out_type=jax.ShapeDtypeStruct((NC, n_nodes, d), jnp.float32),
        mesh=mesh,
        scratch_types=[
            pltpu.VMEM_SHARED((n_nodes, d), jnp.float32),
            pltpu.VMEM((2, 2, CB), jnp.int32),
            pltpu.VMEM((2, CB), jnp.float32),
            pltpu.VMEM((2, CB, d), jnp.float32),
            pltpu.VMEM((2, CB), jnp.int32),
            pltpu.VMEM((rstg, d), jnp.float32),
            pltpu.SemaphoreType.DMA,
            pltpu.SemaphoreType.DMA,
            pltpu.SemaphoreType.DMA,
        ],
    )
    def spmm(feat_hbm, edges_hbm, ev_hbm, out_hbm,
             acc, ebuf, evb, gbuf, sbuf, zstg, esem, gsem, ssem):
        cid = lax.axis_index("c")
        sid = lax.axis_index("s")
        nch = jnp.where(cid == 0, nch0, nch1)
        cbase = jnp.where(cid == 0, sid * nch0, NS * nch0 + sid * nch1)

        zeros16 = jnp.zeros((LANES,), jnp.float32)

        def zb(i, carry):
            for j in range(d // LANES):
                zstg[i, pl.ds(j * LANES, LANES)] = zeros16
            return carry

        lax.fori_loop(0, rstg, zb, 0)
        for off, c in _spans(rows_per_sub):
            start = pl.multiple_of(sid * rows_per_sub + off, 8)
            pltpu.sync_copy(zstg.at[pl.ds(0, c)], acc.at[pl.ds(start, c)])
        if rem:
            @pl.when(sid == NS - 1)
            def _():
                for off, c in _spans(rem):
                    pltpu.sync_copy(
                        zstg.at[pl.ds(0, c)],
                        acc.at[pl.ds(NS * rows_per_sub + off, c)])

        plsc.subcore_barrier()

        pltpu.sync_copy(edges_hbm.at[cbase], ebuf.at[0])
        pltpu.sync_copy(ev_hbm.at[cbase], evb.at[0])
        pltpu.async_copy(feat_hbm.at[ebuf.at[0, 1]], gbuf.at[0], gsem)
        if min(nch0, nch1) > 1:
            pltpu.async_copy(edges_hbm.at[cbase + 1], ebuf.at[1], esem)
            pltpu.async_copy(ev_hbm.at[cbase + 1], evb.at[1], esem)

        def chunk2(it, carry):
            for par in (0, 1):
                ch = it * 2 + par
                nxt = 1 - par

                @pl.when(ch + 1 < nch)
                def _(ch=ch, par=par, nxt=nxt):
                    pltpu.make_async_copy(edges_hbm.at[cbase + ch + 1],
                                          ebuf.at[nxt], esem).wait()
                    pltpu.make_async_copy(ev_hbm.at[cbase + ch + 1],
                                          evb.at[nxt], esem).wait()

                    @pl.when(ch >= 1)
                    def _():
                        pltpu.make_async_copy(
                            gbuf.at[nxt], acc.at[sbuf.at[nxt]], ssem).wait()

                    pltpu.async_copy(feat_hbm.at[ebuf.at[nxt, 1]],
                                     gbuf.at[nxt], gsem)

                pltpu.make_async_copy(feat_hbm.at[ebuf.at[par, 1]],
                                      gbuf.at[par], gsem).wait()

                def scale(g, c2, par=par):
                    wv = evb[par, pl.ds(g * LANES, LANES)]
                    for l in range(LANES):
                        w = wv[l]
                        e = g * LANES + l
                        for j in range(d // LANES):
                            sl = pl.ds(j * LANES, LANES)
                            gbuf[par, e, sl] = gbuf[par, e, sl] * w
                    return c2

                lax.fori_loop(0, CB // LANES, scale, 0)

                for g in range(CB // LANES):
                    sl = pl.ds(g * LANES, LANES)
                    sbuf[par, sl] = ebuf[par, 0, sl]
                pltpu.async_copy(gbuf.at[par], acc.at[sbuf.at[par]], ssem,
                                 add=True)

                @pl.when(ch + 2 < nch)
                def _(ch=ch, par=par):
                    pltpu.async_copy(edges_hbm.at[cbase + ch + 2],
                                     ebuf.at[par], esem)
                    pltpu.async_copy(ev_hbm.at[cbase + ch + 2],
                                     evb.at[par], esem)

            return carry

        lax.fori_loop(0, nch // 2, chunk2, 0)
        for par in (0, 1):
            pltpu.make_async_copy(gbuf.at[par], acc.at[sbuf.at[par]],
                                  ssem).wait()

        plsc.subcore_barrier()

        for off, c in _spans(rows_per_sub):
            start = pl.multiple_of(sid * rows_per_sub + off, 8)
            pltpu.sync_copy(acc.at[pl.ds(start, c)], zstg.at[pl.ds(0, c)])
            pltpu.sync_copy(zstg.at[pl.ds(0, c)],
                            out_hbm.at[cid, pl.ds(start, c)])
        if rem:
            @pl.when(sid == NS - 1)
            def _():
                for off, c in _spans(rem):
                    start = NS * rows_per_sub + off
                    pltpu.sync_copy(acc.at[pl.ds(start, c)],
                                    zstg.at[pl.ds(0, c)])
                    pltpu.sync_copy(zstg.at[pl.ds(0, c)],
                                    out_hbm.at[cid, pl.ds(start, c)])

    return spmm(feat, edges3, ev2)


def _normalize_tc(x):
    n_nodes, d = x.shape
    bm = 2000

    def body(x_ref, o_ref):
        xb = x_ref[...]
        s = jnp.sum(xb, axis=1, keepdims=True) + 0.0001
        o_ref[...] = xb / s

    return pl.pallas_call(
        body,
        grid=(n_nodes // bm,),
        in_specs=[pl.BlockSpec((bm, d), lambda i: (i, 0))],
        out_specs=pl.BlockSpec((bm, d), lambda i: (i, 0)),
        out_shape=jax.ShapeDtypeStruct((n_nodes, d), jnp.float32),
    )(x)


def _fused_linear_tc(partials, w, b, relu):
    _, n_nodes, d = partials.shape
    bm = 2000

    def body(p_ref, w_ref, b_ref, o_ref):
        a = p_ref[0] + p_ref[1]
        y = lax.dot_general(a, w_ref[...], (((1,), (1,)), ((), ())),
                            preferred_element_type=jnp.float32) + b_ref[...]
        o_ref[...] = jnp.maximum(y, 0.0) if relu else y

    return pl.pallas_call(
        body,
        grid=(n_nodes // bm,),
        in_specs=[pl.BlockSpec((NC, bm, d), lambda i: (0, i, 0)),
                  pl.BlockSpec((d, d), lambda i: (0, 0)),
                  pl.BlockSpec((1, d), lambda i: (0, 0))],
        out_specs=pl.BlockSpec((bm, d), lambda i: (i, 0)),
        out_shape=jax.ShapeDtypeStruct((n_nodes, d), jnp.float32),
    )(partials, w, b.reshape(1, d))


def kernel(x, edge_index, edge_values, W1, b1, W2, b2):
    n_nodes, d = x.shape
    n_edges = edge_index.shape[1]
    step = 2 * NS * CB
    e_pad = ((n_edges + step - 1) // step) * step
    row = edge_index[0]
    col = edge_index[1]
    ev = edge_values
    if e_pad != n_edges:
        pad = e_pad - n_edges
        row = jnp.concatenate([row, jnp.zeros((pad,), jnp.int32)])
        col = jnp.concatenate([col, jnp.zeros((pad,), jnp.int32)])
        ev = jnp.concatenate([ev, jnp.zeros((pad,), jnp.float32)])
    t_chunks = e_pad // CB
    nch_sum = t_chunks // NS
    frac0 = 0.48
    nch0 = max(2, int(round(nch_sum * frac0 / 2)) * 2)
    nch1 = nch_sum - nch0
    edges3 = jnp.stack([row.reshape(t_chunks, CB),
                        col.reshape(t_chunks, CB)], axis=1)
    ev2 = ev.reshape(t_chunks, CB)

    xn = _normalize_tc(x)
    p1 = _spmm_sc(xn, edges3, ev2, nch0, nch1)
    h = _fused_linear_tc(p1, W1, b1, relu=True)
    p2 = _spmm_sc(h, edges3, ev2, nch0, nch1)
    y = _fused_linear_tc(p2, W2, b2, relu=False)
    return y


# speedup vs baseline: 1.2754x; 1.0258x over previous
import functools

import jax
import jax.numpy as jnp
from jax import lax
from jax.experimental import pallas as pl
from jax.experimental.pallas import tpu as pltpu
from jax.experimental.pallas import tpu_sc as plsc

NC = 2
NS = 16
LANES = 16
CB = 80


def _spmm_sc(feat, edges3, ev2, nch0, nch1):
    n_nodes, d = feat.shape
    rows_per_sub = (n_nodes // (NS * 8)) * 8
    rem = n_nodes - NS * rows_per_sub
    rstg = 64

    def _spans(length):
        out, off = [], 0
        while off < length:
            c = min(rstg, length - off)
            out.append((off, c))
            off += c
        return out

    mesh = plsc.VectorSubcoreMesh(core_axis_name="c", subcore_axis_name="s")

    @functools.partial(
        pl.kernel,
        out_type=jax.ShapeDtypeStruct((NC, n_nodes, d), jnp.float32),
        mesh=mesh,
        scratch_types=[
            pltpu.VMEM_SHARED((n_nodes, d), jnp.float32),
            pltpu.VMEM((2, 2, CB), jnp.int32),
            pltpu.VMEM((2, CB), jnp.float32),
            pltpu.VMEM((2, CB, d), jnp.float32),
            pltpu.VMEM((2, CB), jnp.int32),
            pltpu.VMEM((rstg, d), jnp.float32),
            pltpu.SemaphoreType.DMA,
            pltpu.SemaphoreType.DMA,
            pltpu.SemaphoreType.DMA,
        ],
    )
    def spmm(feat_hbm, edges_hbm, ev_hbm, out_hbm,
             acc, ebuf, evb, gbuf, sbuf, zstg, esem, gsem, ssem):
        cid = lax.axis_index("c")
        sid = lax.axis_index("s")
        nch = jnp.where(cid == 0, nch0, nch1)
        cbase = jnp.where(cid == 0, sid * nch0, NS * nch0 + sid * nch1)

        zeros16 = jnp.zeros((LANES,), jnp.float32)

        def zb(i, carry):
            for j in range(d // LANES):
                zstg[i, pl.ds(j * LANES, LANES)] = zeros16
            return carry

        lax.fori_loop(0, rstg, zb, 0)
        for off, c in _spans(rows_per_sub):
            start = pl.multiple_of(sid * rows_per_sub + off, 8)
            pltpu.sync_copy(zstg.at[pl.ds(0, c)], acc.at[pl.ds(start, c)])
        if rem:
            @pl.when(sid == NS - 1)
            def _():
                for off, c in _spans(rem):
                    pltpu.sync_copy(
                        zstg.at[pl.ds(0, c)],
                        acc.at[pl.ds(NS * rows_per_sub + off, c)])

        plsc.subcore_barrier()

        pltpu.sync_copy(edges_hbm.at[cbase], ebuf.at[0])
        pltpu.sync_copy(ev_hbm.at[cbase], evb.at[0])
        pltpu.async_copy(feat_hbm.at[ebuf.at[0, 1]], gbuf.at[0], gsem)
        if min(nch0, nch1) > 1:
            pltpu.async_copy(edges_hbm.at[cbase + 1], ebuf.at[1], esem)
            pltpu.async_copy(ev_hbm.at[cbase + 1], evb.at[1], esem)

        def chunk2(it, carry):
            for par in (0, 1):
                ch = it * 2 + par
                nxt = 1 - par

                @pl.when(ch + 1 < nch)
                def _(ch=ch, par=par, nxt=nxt):
                    pltpu.make_async_copy(edges_hbm.at[cbase + ch + 1],
                                          ebuf.at[nxt], esem).wait()
                    pltpu.make_async_copy(ev_hbm.at[cbase + ch + 1],
                                          evb.at[nxt], esem).wait()

                    @pl.when(ch >= 1)
                    def _():
                        pltpu.make_async_copy(
                            gbuf.at[nxt], acc.at[sbuf.at[nxt]], ssem).wait()

                    pltpu.async_copy(feat_hbm.at[ebuf.at[nxt, 1]],
                                     gbuf.at[nxt], gsem)

                pltpu.make_async_copy(feat_hbm.at[ebuf.at[par, 1]],
                                      gbuf.at[par], gsem).wait()

                def scale(g, c2, par=par):
                    wv = evb[par, pl.ds(g * LANES, LANES)]
                    for l in range(LANES):
                        w = wv[l]
                        e = g * LANES + l
                        for j in range(d // LANES):
                            sl = pl.ds(j * LANES, LANES)
                            gbuf[par, e, sl] = gbuf[par, e, sl] * w
                    return c2

                lax.fori_loop(0, CB // LANES, scale, 0)

                for g in range(CB // LANES):
                    sl = pl.ds(g * LANES, LANES)
                    sbuf[par, sl] = ebuf[par, 0, sl]
                pltpu.async_copy(gbuf.at[par], acc.at[sbuf.at[par]], ssem,
                                 add=True)

                @pl.when(ch + 2 < nch)
                def _(ch=ch, par=par):
                    pltpu.async_copy(edges_hbm.at[cbase + ch + 2],
                                     ebuf.at[par], esem)
                    pltpu.async_copy(ev_hbm.at[cbase + ch + 2],
                                     evb.at[par], esem)

            return carry

        lax.fori_loop(0, nch // 2, chunk2, 0)
        for par in (0, 1):
            pltpu.make_async_copy(gbuf.at[par], acc.at[sbuf.at[par]],
                                  ssem).wait()

        plsc.subcore_barrier()

        for off, c in _spans(rows_per_sub):
            start = pl.multiple_of(sid * rows_per_sub + off, 8)
            pltpu.sync_copy(acc.at[pl.ds(start, c)], zstg.at[pl.ds(0, c)])
            pltpu.sync_copy(zstg.at[pl.ds(0, c)],
                            out_hbm.at[cid, pl.ds(start, c)])
        if rem:
            @pl.when(sid == NS - 1)
            def _():
                for off, c in _spans(rem):
                    start = NS * rows_per_sub + off
                    pltpu.sync_copy(acc.at[pl.ds(start, c)],
                                    zstg.at[pl.ds(0, c)])
                    pltpu.sync_copy(zstg.at[pl.ds(0, c)],
                                    out_hbm.at[cid, pl.ds(start, c)])

    return spmm(feat, edges3, ev2)


def _normalize_tc(x):
    n_nodes, d = x.shape
    bm = 2000

    def body(x_ref, o_ref):
        xb = x_ref[...]
        s = jnp.sum(xb, axis=1, keepdims=True) + 0.0001
        o_ref[...] = xb / s

    return pl.pallas_call(
        body,
        grid=(n_nodes // bm,),
        in_specs=[pl.BlockSpec((bm, d), lambda i: (i, 0))],
        out_specs=pl.BlockSpec((bm, d), lambda i: (i, 0)),
        out_shape=jax.ShapeDtypeStruct((n_nodes, d), jnp.float32),
    )(x)


def _fused_linear_tc(partials, w, b, relu):
    _, n_nodes, d = partials.shape
    bm = 2000

    def body(p_ref, w_ref, b_ref, o_ref):
        a = p_ref[0] + p_ref[1]
        y = lax.dot_general(a, w_ref[...], (((1,), (1,)), ((), ())),
                            preferred_element_type=jnp.float32) + b_ref[...]
        o_ref[...] = jnp.maximum(y, 0.0) if relu else y

    return pl.pallas_call(
        body,
        grid=(n_nodes // bm,),
        in_specs=[pl.BlockSpec((NC, bm, d), lambda i: (0, i, 0)),
                  pl.BlockSpec((d, d), lambda i: (0, 0)),
                  pl.BlockSpec((1, d), lambda i: (0, 0))],
        out_specs=pl.BlockSpec((bm, d), lambda i: (i, 0)),
        out_shape=jax.ShapeDtypeStruct((n_nodes, d), jnp.float32),
    )(partials, w, b.reshape(1, d))


def kernel(x, edge_index, edge_values, W1, b1, W2, b2):
    n_nodes, d = x.shape
    n_edges = edge_index.shape[1]
    step = 2 * NS * CB
    e_pad = ((n_edges + step - 1) // step) * step
    row = edge_index[0]
    col = edge_index[1]
    ev = edge_values
    if e_pad != n_edges:
        pad = e_pad - n_edges
        row = jnp.concatenate([row, jnp.zeros((pad,), jnp.int32)])
        col = jnp.concatenate([col, jnp.zeros((pad,), jnp.int32)])
        ev = jnp.concatenate([ev, jnp.zeros((pad,), jnp.float32)])
    t_chunks = e_pad // CB
    nch_sum = t_chunks // NS
    frac0 = 0.50
    nch0 = max(2, int(round(nch_sum * frac0 / 2)) * 2)
    nch1 = nch_sum - nch0
    edges3 = jnp.stack([row.reshape(t_chunks, CB),
                        col.reshape(t_chunks, CB)], axis=1)
    ev2 = ev.reshape(t_chunks, CB)

    xn = _normalize_tc(x)
    p1 = _spmm_sc(xn, edges3, ev2, nch0, nch1)
    h = _fused_linear_tc(p1, W1, b1, relu=True)
    p2 = _spmm_sc(h, edges3, ev2, nch0, nch1)
    y = _fused_linear_tc(p2, W2, b2, relu=False)
    return y
